# baseline (speedup 1.0000x reference)
import functools

import jax
import jax.numpy as jnp
from jax import lax
from jax.experimental import pallas as pl
from jax.experimental.pallas import tpu as pltpu
from jax.experimental.pallas import tpu_sc as plsc

B = 16384
D_ART = 32
EPS = 1e-3
N_G, N_GR, N_C, N_S = 21, 31, 21, 57
OFF_GR = N_G
OFF_C = N_G + N_GR
OFF_S = N_G + N_GR + N_C
P_ROWS = N_G + N_GR + N_C + N_S
P_PAD = 136
D_GRP, D_GRA, D_COL, D_SEC = 10, 15, 10, 14


def _sc_gather(article_id, emb_table, lut_g, lut_s, lut_gr, lut_c):
    info = plsc.get_sparse_core_info()
    nc, ns = info.num_cores, info.num_subcores
    nw = nc * ns
    bpw = B // nw
    nchunk = bpw // 128

    mesh = plsc.VectorSubcoreMesh(core_axis_name="c", subcore_axis_name="s",
                                  num_cores=nc)

    @functools.partial(
        pl.kernel,
        mesh=mesh,
        out_type=[
            jax.ShapeDtypeStruct((D_ART, B), jnp.float32),
            jax.ShapeDtypeStruct((B,), jnp.int32),
            jax.ShapeDtypeStruct((B,), jnp.int32),
            jax.ShapeDtypeStruct((B,), jnp.int32),
            jax.ShapeDtypeStruct((B,), jnp.int32),
        ],
        scratch_types=[
            pltpu.VMEM((nchunk, 128), jnp.int32),
            pltpu.VMEM((nchunk, D_ART, 128), jnp.int32),
            pltpu.VMEM((D_ART, bpw), jnp.float32),
            pltpu.VMEM((bpw,), jnp.int32),
            pltpu.VMEM((bpw,), jnp.int32),
            pltpu.VMEM((bpw,), jnp.int32),
            pltpu.VMEM((bpw,), jnp.int32),
            pltpu.SemaphoreType.DMA,
        ],
        compiler_params=pltpu.CompilerParams(use_tc_tiling_on_sc=False),
    )
    def k(aid_hbm, idx3_hbm, emb_hbm, lg_hbm, ls_hbm, lgr_hbm, lc_hbm,
          emb_out, g_out, s_out, gr_out, c_out,
          idx_v, idx3_v, rows_v, g_v, s_v, gr_v, c_v, sem):
        wid = lax.axis_index("s") * nc + lax.axis_index("c")
        base = wid * bpw
        pltpu.sync_copy(aid_hbm.at[pl.ds(wid * nchunk, nchunk), :], idx_v)
        pltpu.sync_copy(idx3_hbm.at[pl.ds(wid * nchunk, nchunk), :, :], idx3_v)
        copies = []
        for j in range(nchunk):
            for d in range(D_ART):
                copies.append(pltpu.async_copy(
                    emb_hbm.at[idx3_v.at[j, d]],
                    rows_v.at[d, pl.ds(j * 128, 128)], sem))
            idx = idx_v.at[j]
            lo = j * 128
            copies.append(pltpu.async_copy(
                lg_hbm.at[idx], g_v.at[pl.ds(lo, 128)], sem))
            copies.append(pltpu.async_copy(
                ls_hbm.at[idx], s_v.at[pl.ds(lo, 128)], sem))
            copies.append(pltpu.async_copy(
                lgr_hbm.at[idx], gr_v.at[pl.ds(lo, 128)], sem))
            copies.append(pltpu.async_copy(
                lc_hbm.at[idx], c_v.at[pl.ds(lo, 128)], sem))
        for cp in copies:
            cp.wait()
        pltpu.sync_copy(rows_v, emb_out.at[:, pl.ds(base, bpw)])
        pltpu.sync_copy(g_v, g_out.at[pl.ds(base, bpw)])
        pltpu.sync_copy(s_v, s_out.at[pl.ds(base, bpw)])
        pltpu.sync_copy(gr_v, gr_out.at[pl.ds(base, bpw)])
        pltpu.sync_copy(c_v, c_out.at[pl.ds(base, bpw)])

    v = emb_table.shape[0]
    idx3 = (article_id.reshape(B // 128, 1, 128)
            + (jnp.arange(D_ART, dtype=jnp.int32) * v).reshape(1, D_ART, 1))
    return k(article_id.reshape(B // 128, 128), idx3,
             emb_table.T.reshape(-1),
             lut_g, lut_s, lut_gr, lut_c)


def _prep_kernel(gt_ref, st_ref, grt_ref, ct_ref,
                 gam_ref, bet_ref, mu_ref, var_ref, w_ref,
                 w1_ref, p_ref, bias_ref):
    gam = gam_ref[:, :]
    a = gam * lax.rsqrt(var_ref[:, :] + EPS)
    bcol = bet_ref[:, :] - mu_ref[:, :] * a
    w = w_ref[:, :]
    wp = w * a
    bias_ref[:, :] = jnp.sum(w * bcol, axis=0, keepdims=True)
    w1_ref[:, :] = wp[0:D_ART, :]
    o0 = D_ART
    pg = jnp.dot(gt_ref[:, :], wp[o0:o0 + D_GRP, :],
                 preferred_element_type=jnp.float32)
    o1 = o0 + D_GRP
    pgr = jnp.dot(grt_ref[:, :], wp[o1:o1 + D_GRA, :],
                  preferred_element_type=jnp.float32)
    o2 = o1 + D_GRA
    pc = jnp.dot(ct_ref[:, :], wp[o2:o2 + D_COL, :],
                 preferred_element_type=jnp.float32)
    o3 = o2 + D_COL
    ps = jnp.dot(st_ref[:, :], wp[o3:o3 + D_SEC, :],
                 preferred_element_type=jnp.float32)
    pad = jnp.zeros((P_PAD - P_ROWS, 128), jnp.float32)
    p_ref[:, :] = jnp.concatenate([pg, pgr, pc, ps, pad], axis=0)


def _main_kernel(emb_ref, g_ref, s_ref, gr_ref, c_ref,
                 w1_ref, p_ref, bias_ref, out_ref):
    bb = emb_ref.shape[1]
    gid = g_ref[0]
    sid = s_ref[0]
    grid = gr_ref[0]
    cid = c_ref[0]
    col = lax.broadcasted_iota(jnp.int32, (bb, P_PAD), 1)
    oh = ((col == gid).astype(jnp.float32)
          + (col == grid + OFF_GR).astype(jnp.float32)
          + (col == cid + OFF_C).astype(jnp.float32)
          + (col == sid + OFF_S).astype(jnp.float32))
    acc = lax.dot_general(emb_ref[:, :], w1_ref[:, :],
                          dimension_numbers=(((0,), (0,)), ((), ())),
                          preferred_element_type=jnp.float32)
    acc = acc + jnp.dot(oh, p_ref[:, :], preferred_element_type=jnp.float32)
    out_ref[:, :] = acc + bias_ref[:, :]


def kernel(article_id, emb_table, group_table, section_table, graphical_table,
           colour_table, lut_group, lut_section, lut_graphical, lut_colour,
           gamma, beta, moving_mean, moving_var, W):
    embt, gid, sid, grid, cid = _sc_gather(
        article_id, emb_table,
        lut_group, lut_section, lut_graphical, lut_colour)

    cdim = W.shape[0]
    w1, p, bias = pl.pallas_call(
        _prep_kernel,
        out_shape=[
            jax.ShapeDtypeStruct((D_ART, 128), jnp.float32),
            jax.ShapeDtypeStruct((P_PAD, 128), jnp.float32),
            jax.ShapeDtypeStruct((1, 128), jnp.float32),
        ],
    )(group_table, section_table, graphical_table, colour_table,
      gamma.reshape(cdim, 1), beta.reshape(cdim, 1),
      moving_mean.reshape(cdim, 1), moving_var.reshape(cdim, 1), W)

    bb = 512
    nblk = B // bb
    ids_spec = pl.BlockSpec((1, bb, 1), lambda i: (i, 0, 0))
    full = lambda shape: pl.BlockSpec(shape, lambda i: (0,) * len(shape))
    out = pl.pallas_call(
        _main_kernel,
        grid=(nblk,),
        in_specs=[
            pl.BlockSpec((D_ART, bb), lambda i: (0, i)),
            ids_spec, ids_spec, ids_spec, ids_spec,
            full((D_ART, 128)), full((P_PAD, 128)), full((1, 128)),
        ],
        out_specs=pl.BlockSpec((bb, 128), lambda i: (i, 0)),
        out_shape=jax.ShapeDtypeStruct((B, 128), jnp.float32),
    )(embt,
      gid.reshape(nblk, bb, 1), sid.reshape(nblk, bb, 1),
      grid.reshape(nblk, bb, 1), cid.reshape(nblk, bb, 1),
      w1, p, bias)
    return out


# speedup vs baseline: 10.6995x; 10.6995x over previous
import functools

import jax
import jax.numpy as jnp
from jax import lax
from jax.experimental import pallas as pl
from jax.experimental.pallas import tpu as pltpu
from jax.experimental.pallas import tpu_sc as plsc

B = 16384
D_ART = 32
EPS = 1e-3
N_G, N_GR, N_C, N_S = 21, 31, 21, 57
OFF_GR = N_G
OFF_C = N_G + N_GR
OFF_S = N_G + N_GR + N_C
P_ROWS = N_G + N_GR + N_C + N_S
P_PAD = 136
D_GRP, D_GRA, D_COL, D_SEC = 10, 15, 10, 14


FLAT_L = 32768


def _flatten_kernel(in_ref, out_ref):
    blk = in_ref[...]
    out_ref[...] = jnp.reshape(blk, (blk.shape[0] * blk.shape[1],))


def _tc_flatten(emb_table):
    v, d_art = emb_table.shape
    nj = -(-v // FLAT_L)
    flat = pl.pallas_call(
        _flatten_kernel,
        grid=(d_art // 8, nj),
        in_specs=[pl.BlockSpec((8, FLAT_L), lambda g, j: (g, j))],
        out_specs=pl.BlockSpec((8 * FLAT_L,), lambda g, j: (g * nj + j,)),
        out_shape=jax.ShapeDtypeStruct((d_art * nj * FLAT_L,), jnp.float32),
    )(emb_table.T)
    return flat, nj


def _sc_gather(article_id, emb_table, lut_g, lut_s, lut_gr, lut_c):
    info = plsc.get_sparse_core_info()
    nc, ns = info.num_cores, info.num_subcores
    nw = nc * ns
    bpw = B // nw
    nchunk = bpw // 128

    mesh = plsc.VectorSubcoreMesh(core_axis_name="c", subcore_axis_name="s",
                                  num_cores=nc)

    @functools.partial(
        pl.kernel,
        mesh=mesh,
        out_type=[
            jax.ShapeDtypeStruct((D_ART, B), jnp.float32),
            jax.ShapeDtypeStruct((B,), jnp.int32),
            jax.ShapeDtypeStruct((B,), jnp.int32),
            jax.ShapeDtypeStruct((B,), jnp.int32),
            jax.ShapeDtypeStruct((B,), jnp.int32),
        ],
        scratch_types=[
            pltpu.VMEM((nchunk, 128), jnp.int32),
            pltpu.VMEM((nchunk, D_ART, 128), jnp.int32),
            pltpu.VMEM((D_ART, bpw), jnp.float32),
            pltpu.VMEM((bpw,), jnp.int32),
            pltpu.VMEM((bpw,), jnp.int32),
            pltpu.VMEM((bpw,), jnp.int32),
            pltpu.VMEM((bpw,), jnp.int32),
            pltpu.SemaphoreType.DMA,
        ],
        compiler_params=pltpu.CompilerParams(use_tc_tiling_on_sc=False),
    )
    def k(aid_hbm, idx3_hbm, emb_hbm, lg_hbm, ls_hbm, lgr_hbm, lc_hbm,
          emb_out, g_out, s_out, gr_out, c_out,
          idx_v, idx3_v, rows_v, g_v, s_v, gr_v, c_v, sem):
        wid = lax.axis_index("s") * nc + lax.axis_index("c")
        base = wid * bpw
        pltpu.sync_copy(aid_hbm.at[pl.ds(wid * nchunk, nchunk), :], idx_v)
        pltpu.sync_copy(idx3_hbm.at[pl.ds(wid * nchunk, nchunk), :, :], idx3_v)
        copies = []
        for j in range(nchunk):
            for d in range(D_ART):
                copies.append(pltpu.async_copy(
                    emb_hbm.at[idx3_v.at[j, d]],
                    rows_v.at[d, pl.ds(j * 128, 128)], sem))
            idx = idx_v.at[j]
            lo = j * 128
            copies.append(pltpu.async_copy(
                lg_hbm.at[idx], g_v.at[pl.ds(lo, 128)], sem))
            copies.append(pltpu.async_copy(
                ls_hbm.at[idx], s_v.at[pl.ds(lo, 128)], sem))
            copies.append(pltpu.async_copy(
                lgr_hbm.at[idx], gr_v.at[pl.ds(lo, 128)], sem))
            copies.append(pltpu.async_copy(
                lc_hbm.at[idx], c_v.at[pl.ds(lo, 128)], sem))
        for cp in copies:
            cp.wait()
        pltpu.sync_copy(rows_v, emb_out.at[:, pl.ds(base, bpw)])
        pltpu.sync_copy(g_v, g_out.at[pl.ds(base, bpw)])
        pltpu.sync_copy(s_v, s_out.at[pl.ds(base, bpw)])
        pltpu.sync_copy(gr_v, gr_out.at[pl.ds(base, bpw)])
        pltpu.sync_copy(c_v, c_out.at[pl.ds(base, bpw)])

    emb_flat, nj = _tc_flatten(emb_table)
    d = jnp.arange(D_ART, dtype=jnp.int32)
    c_d = (d // 8) * (8 * nj * FLAT_L) + (d % 8) * FLAT_L
    t_a = (article_id // FLAT_L) * (8 * FLAT_L) + article_id % FLAT_L
    idx3 = (t_a.reshape(B // 128, 1, 128) + c_d.reshape(1, D_ART, 1))
    return k(article_id.reshape(B // 128, 128), idx3, emb_flat,
             lut_g, lut_s, lut_gr, lut_c)


def _prep_kernel(gt_ref, st_ref, grt_ref, ct_ref,
                 gam_ref, bet_ref, mu_ref, var_ref, w_ref,
                 w1_ref, p_ref, bias_ref):
    gam = gam_ref[:, :]
    a = gam * lax.rsqrt(var_ref[:, :] + EPS)
    bcol = bet_ref[:, :] - mu_ref[:, :] * a
    w = w_ref[:, :]
    wp = w * a
    bias_ref[:, :] = jnp.sum(w * bcol, axis=0, keepdims=True)
    w1_ref[:, :] = wp[0:D_ART, :]
    o0 = D_ART
    pg = jnp.dot(gt_ref[:, :], wp[o0:o0 + D_GRP, :],
                 preferred_element_type=jnp.float32)
    o1 = o0 + D_GRP
    pgr = jnp.dot(grt_ref[:, :], wp[o1:o1 + D_GRA, :],
                  preferred_element_type=jnp.float32)
    o2 = o1 + D_GRA
    pc = jnp.dot(ct_ref[:, :], wp[o2:o2 + D_COL, :],
                 preferred_element_type=jnp.float32)
    o3 = o2 + D_COL
    ps = jnp.dot(st_ref[:, :], wp[o3:o3 + D_SEC, :],
                 preferred_element_type=jnp.float32)
    pad = jnp.zeros((P_PAD - P_ROWS, 128), jnp.float32)
    p_ref[:, :] = jnp.concatenate([pg, pgr, pc, ps, pad], axis=0)


def _main_kernel(emb_ref, g_ref, s_ref, gr_ref, c_ref,
                 w1_ref, p_ref, bias_ref, out_ref):
    bb = emb_ref.shape[1]
    gid = g_ref[0]
    sid = s_ref[0]
    grid = gr_ref[0]
    cid = c_ref[0]
    col = lax.broadcasted_iota(jnp.int32, (bb, P_PAD), 1)
    oh = ((col == gid).astype(jnp.float32)
          + (col == grid + OFF_GR).astype(jnp.float32)
          + (col == cid + OFF_C).astype(jnp.float32)
          + (col == sid + OFF_S).astype(jnp.float32))
    acc = lax.dot_general(emb_ref[:, :], w1_ref[:, :],
                          dimension_numbers=(((0,), (0,)), ((), ())),
                          preferred_element_type=jnp.float32)
    acc = acc + jnp.dot(oh, p_ref[:, :], preferred_element_type=jnp.float32)
    out_ref[:, :] = acc + bias_ref[:, :]


def kernel(article_id, emb_table, group_table, section_table, graphical_table,
           colour_table, lut_group, lut_section, lut_graphical, lut_colour,
           gamma, beta, moving_mean, moving_var, W):
    embt, gid, sid, grid, cid = _sc_gather(
        article_id, emb_table,
        lut_group, lut_section, lut_graphical, lut_colour)

    cdim = W.shape[0]
    w1, p, bias = pl.pallas_call(
        _prep_kernel,
        out_shape=[
            jax.ShapeDtypeStruct((D_ART, 128), jnp.float32),
            jax.ShapeDtypeStruct((P_PAD, 128), jnp.float32),
            jax.ShapeDtypeStruct((1, 128), jnp.float32),
        ],
    )(group_table, section_table, graphical_table, colour_table,
      gamma.reshape(cdim, 1), beta.reshape(cdim, 1),
      moving_mean.reshape(cdim, 1), moving_var.reshape(cdim, 1), W)

    bb = 512
    nblk = B // bb
    ids_spec = pl.BlockSpec((1, bb, 1), lambda i: (i, 0, 0))
    full = lambda shape: pl.BlockSpec(shape, lambda i: (0,) * len(shape))
    out = pl.pallas_call(
        _main_kernel,
        grid=(nblk,),
        in_specs=[
            pl.BlockSpec((D_ART, bb), lambda i: (0, i)),
            ids_spec, ids_spec, ids_spec, ids_spec,
            full((D_ART, 128)), full((P_PAD, 128)), full((1, 128)),
        ],
        out_specs=pl.BlockSpec((bb, 128), lambda i: (i, 0)),
        out_shape=jax.ShapeDtypeStruct((B, 128), jnp.float32),
    )(embt,
      gid.reshape(nblk, bb, 1), sid.reshape(nblk, bb, 1),
      grid.reshape(nblk, bb, 1), cid.reshape(nblk, bb, 1),
      w1, p, bias)
    return out


# speedup vs baseline: 12.2566x; 1.1455x over previous
import functools

import jax
import jax.numpy as jnp
from jax import lax
from jax.experimental import pallas as pl
from jax.experimental.pallas import tpu as pltpu
from jax.experimental.pallas import tpu_sc as plsc

B = 16384
D_ART = 32
EPS = 1e-3
N_G, N_GR, N_C, N_S = 21, 31, 21, 57
OFF_GR = N_G
OFF_C = N_G + N_GR
OFF_S = N_G + N_GR + N_C
P_ROWS = N_G + N_GR + N_C + N_S
P_PAD = 136
D_GRP, D_GRA, D_COL, D_SEC = 10, 15, 10, 14


FLAT_L = 65536


def _flatten_kernel(in_ref, out_ref):
    blk = in_ref[...]
    out_ref[...] = jnp.reshape(blk, (blk.shape[0] * blk.shape[1],))


def _tc_flatten(emb_table):
    v, d_art = emb_table.shape
    nj = -(-v // FLAT_L)
    flat = pl.pallas_call(
        _flatten_kernel,
        grid=(d_art // 8, nj),
        in_specs=[pl.BlockSpec((8, FLAT_L), lambda g, j: (g, j))],
        out_specs=pl.BlockSpec((8 * FLAT_L,), lambda g, j: (g * nj + j,)),
        out_shape=jax.ShapeDtypeStruct((d_art * nj * FLAT_L,), jnp.float32),
    )(emb_table.T)
    return flat, nj


def _sc_gather(article_id, emb_table, lut_g, lut_s, lut_gr, lut_c):
    info = plsc.get_sparse_core_info()
    nc, ns = info.num_cores, info.num_subcores
    nw = nc * ns
    bpw = B // nw
    nchunk = bpw // 128

    mesh = plsc.VectorSubcoreMesh(core_axis_name="c", subcore_axis_name="s",
                                  num_cores=nc)

    @functools.partial(
        pl.kernel,
        mesh=mesh,
        out_type=[
            jax.ShapeDtypeStruct((D_ART, B), jnp.float32),
            jax.ShapeDtypeStruct((B,), jnp.int32),
            jax.ShapeDtypeStruct((B,), jnp.int32),
            jax.ShapeDtypeStruct((B,), jnp.int32),
            jax.ShapeDtypeStruct((B,), jnp.int32),
        ],
        scratch_types=[
            pltpu.VMEM((nchunk, 128), jnp.int32),
            pltpu.VMEM((nchunk, D_ART, 128), jnp.int32),
            pltpu.VMEM((D_ART, bpw), jnp.float32),
            pltpu.VMEM((bpw,), jnp.int32),
            pltpu.VMEM((bpw,), jnp.int32),
            pltpu.VMEM((bpw,), jnp.int32),
            pltpu.VMEM((bpw,), jnp.int32),
            pltpu.SemaphoreType.DMA,
        ],
        compiler_params=pltpu.CompilerParams(use_tc_tiling_on_sc=False),
    )
    def k(aid_hbm, idx3_hbm, emb_hbm, lg_hbm, ls_hbm, lgr_hbm, lc_hbm,
          emb_out, g_out, s_out, gr_out, c_out,
          idx_v, idx3_v, rows_v, g_v, s_v, gr_v, c_v, sem):
        wid = lax.axis_index("s") * nc + lax.axis_index("c")
        base = wid * bpw
        pltpu.sync_copy(aid_hbm.at[pl.ds(wid * nchunk, nchunk), :], idx_v)
        pltpu.sync_copy(idx3_hbm.at[pl.ds(wid * nchunk, nchunk), :, :], idx3_v)
        copies = []
        for j in range(nchunk):
            for d in range(D_ART):
                copies.append(pltpu.async_copy(
                    emb_hbm.at[idx3_v.at[j, d]],
                    rows_v.at[d, pl.ds(j * 128, 128)], sem))
            idx = idx_v.at[j]
            lo = j * 128
            copies.append(pltpu.async_copy(
                lg_hbm.at[idx], g_v.at[pl.ds(lo, 128)], sem))
            copies.append(pltpu.async_copy(
                ls_hbm.at[idx], s_v.at[pl.ds(lo, 128)], sem))
            copies.append(pltpu.async_copy(
                lgr_hbm.at[idx], gr_v.at[pl.ds(lo, 128)], sem))
            copies.append(pltpu.async_copy(
                lc_hbm.at[idx], c_v.at[pl.ds(lo, 128)], sem))
        for cp in copies:
            cp.wait()
        pltpu.sync_copy(rows_v, emb_out.at[:, pl.ds(base, bpw)])
        pltpu.sync_copy(g_v, g_out.at[pl.ds(base, bpw)])
        pltpu.sync_copy(s_v, s_out.at[pl.ds(base, bpw)])
        pltpu.sync_copy(gr_v, gr_out.at[pl.ds(base, bpw)])
        pltpu.sync_copy(c_v, c_out.at[pl.ds(base, bpw)])

    emb_flat, nj = _tc_flatten(emb_table)
    d = jnp.arange(D_ART, dtype=jnp.int32)
    c_d = (d // 8) * (8 * nj * FLAT_L) + (d % 8) * FLAT_L
    t_a = (article_id // FLAT_L) * (8 * FLAT_L) + article_id % FLAT_L
    idx3 = (t_a.reshape(B // 128, 1, 128) + c_d.reshape(1, D_ART, 1))
    return k(article_id.reshape(B // 128, 128), idx3, emb_flat,
             lut_g, lut_s, lut_gr, lut_c)


def _prep_kernel(gt_ref, st_ref, grt_ref, ct_ref,
                 gam_ref, bet_ref, mu_ref, var_ref, w_ref,
                 w1_ref, p_ref, bias_ref):
    gam = gam_ref[:, :]
    a = gam * lax.rsqrt(var_ref[:, :] + EPS)
    bcol = bet_ref[:, :] - mu_ref[:, :] * a
    w = w_ref[:, :]
    wp = w * a
    bias_ref[:, :] = jnp.sum(w * bcol, axis=0, keepdims=True)
    w1_ref[:, :] = wp[0:D_ART, :]
    o0 = D_ART
    pg = jnp.dot(gt_ref[:, :], wp[o0:o0 + D_GRP, :],
                 preferred_element_type=jnp.float32)
    o1 = o0 + D_GRP
    pgr = jnp.dot(grt_ref[:, :], wp[o1:o1 + D_GRA, :],
                  preferred_element_type=jnp.float32)
    o2 = o1 + D_GRA
    pc = jnp.dot(ct_ref[:, :], wp[o2:o2 + D_COL, :],
                 preferred_element_type=jnp.float32)
    o3 = o2 + D_COL
    ps = jnp.dot(st_ref[:, :], wp[o3:o3 + D_SEC, :],
                 preferred_element_type=jnp.float32)
    pad = jnp.zeros((P_PAD - P_ROWS, 128), jnp.float32)
    p_ref[:, :] = jnp.concatenate([pg, pgr, pc, ps, pad], axis=0)


def _main_kernel(emb_ref, g_ref, s_ref, gr_ref, c_ref,
                 w1_ref, p_ref, bias_ref, out_ref):
    bb = emb_ref.shape[1]
    gid = g_ref[0]
    sid = s_ref[0]
    grid = gr_ref[0]
    cid = c_ref[0]
    col = lax.broadcasted_iota(jnp.int32, (bb, P_PAD), 1)
    oh = ((col == gid).astype(jnp.float32)
          + (col == grid + OFF_GR).astype(jnp.float32)
          + (col == cid + OFF_C).astype(jnp.float32)
          + (col == sid + OFF_S).astype(jnp.float32))
    acc = lax.dot_general(emb_ref[:, :], w1_ref[:, :],
                          dimension_numbers=(((0,), (0,)), ((), ())),
                          preferred_element_type=jnp.float32)
    acc = acc + jnp.dot(oh, p_ref[:, :], preferred_element_type=jnp.float32)
    out_ref[:, :] = acc + bias_ref[:, :]


def kernel(article_id, emb_table, group_table, section_table, graphical_table,
           colour_table, lut_group, lut_section, lut_graphical, lut_colour,
           gamma, beta, moving_mean, moving_var, W):
    embt, gid, sid, grid, cid = _sc_gather(
        article_id, emb_table,
        lut_group, lut_section, lut_graphical, lut_colour)

    cdim = W.shape[0]
    w1, p, bias = pl.pallas_call(
        _prep_kernel,
        out_shape=[
            jax.ShapeDtypeStruct((D_ART, 128), jnp.float32),
            jax.ShapeDtypeStruct((P_PAD, 128), jnp.float32),
            jax.ShapeDtypeStruct((1, 128), jnp.float32),
        ],
    )(group_table, section_table, graphical_table, colour_table,
      gamma.reshape(cdim, 1), beta.reshape(cdim, 1),
      moving_mean.reshape(cdim, 1), moving_var.reshape(cdim, 1), W)

    bb = 512
    nblk = B // bb
    ids_spec = pl.BlockSpec((1, bb, 1), lambda i: (i, 0, 0))
    full = lambda shape: pl.BlockSpec(shape, lambda i: (0,) * len(shape))
    out = pl.pallas_call(
        _main_kernel,
        grid=(nblk,),
        in_specs=[
            pl.BlockSpec((D_ART, bb), lambda i: (0, i)),
            ids_spec, ids_spec, ids_spec, ids_spec,
            full((D_ART, 128)), full((P_PAD, 128)), full((1, 128)),
        ],
        out_specs=pl.BlockSpec((bb, 128), lambda i: (i, 0)),
        out_shape=jax.ShapeDtypeStruct((B, 128), jnp.float32),
    )(embt,
      gid.reshape(nblk, bb, 1), sid.reshape(nblk, bb, 1),
      grid.reshape(nblk, bb, 1), cid.reshape(nblk, bb, 1),
      w1, p, bias)
    return out


# speedup vs baseline: 13.0294x; 1.0631x over previous
import functools

import jax
import jax.numpy as jnp
from jax import lax
from jax.experimental import pallas as pl
from jax.experimental.pallas import tpu as pltpu
from jax.experimental.pallas import tpu_sc as plsc

B = 16384
D_ART = 32
EPS = 1e-3
N_G, N_GR, N_C, N_S = 21, 31, 21, 57
OFF_GR = N_G
OFF_C = N_G + N_GR
OFF_S = N_G + N_GR + N_C
P_ROWS = N_G + N_GR + N_C + N_S
P_PAD = 136
D_GRP, D_GRA, D_COL, D_SEC = 10, 15, 10, 14


FLAT_L = 131072


def _flatten_kernel(in_ref, out_ref):
    blk = in_ref[...]
    out_ref[...] = jnp.reshape(blk, (blk.shape[0] * blk.shape[1],))


def _tc_flatten(emb_table):
    v, d_art = emb_table.shape
    nj = -(-v // FLAT_L)
    flat = pl.pallas_call(
        _flatten_kernel,
        grid=(d_art // 8, nj),
        in_specs=[pl.BlockSpec((8, FLAT_L), lambda g, j: (g, j))],
        out_specs=pl.BlockSpec((8 * FLAT_L,), lambda g, j: (g * nj + j,)),
        out_shape=jax.ShapeDtypeStruct((d_art * nj * FLAT_L,), jnp.float32),
    )(emb_table.T)
    return flat, nj


def _sc_gather(article_id, emb_table, lut_g, lut_s, lut_gr, lut_c):
    info = plsc.get_sparse_core_info()
    nc, ns = info.num_cores, info.num_subcores
    nw = nc * ns
    bpw = B // nw
    nchunk = bpw // 128

    mesh = plsc.VectorSubcoreMesh(core_axis_name="c", subcore_axis_name="s",
                                  num_cores=nc)

    @functools.partial(
        pl.kernel,
        mesh=mesh,
        out_type=[
            jax.ShapeDtypeStruct((D_ART, B), jnp.float32),
            jax.ShapeDtypeStruct((B,), jnp.int32),
            jax.ShapeDtypeStruct((B,), jnp.int32),
            jax.ShapeDtypeStruct((B,), jnp.int32),
            jax.ShapeDtypeStruct((B,), jnp.int32),
        ],
        scratch_types=[
            pltpu.VMEM((nchunk, 128), jnp.int32),
            pltpu.VMEM((nchunk, D_ART, 128), jnp.int32),
            pltpu.VMEM((D_ART, bpw), jnp.float32),
            pltpu.VMEM((bpw,), jnp.int32),
            pltpu.VMEM((bpw,), jnp.int32),
            pltpu.VMEM((bpw,), jnp.int32),
            pltpu.VMEM((bpw,), jnp.int32),
            pltpu.SemaphoreType.DMA,
        ],
        compiler_params=pltpu.CompilerParams(use_tc_tiling_on_sc=False),
    )
    def k(aid_hbm, idx3_hbm, emb_hbm, lg_hbm, ls_hbm, lgr_hbm, lc_hbm,
          emb_out, g_out, s_out, gr_out, c_out,
          idx_v, idx3_v, rows_v, g_v, s_v, gr_v, c_v, sem):
        wid = lax.axis_index("s") * nc + lax.axis_index("c")
        base = wid * bpw
        pltpu.sync_copy(aid_hbm.at[pl.ds(wid * nchunk, nchunk), :], idx_v)
        pltpu.sync_copy(idx3_hbm.at[pl.ds(wid * nchunk, nchunk), :, :], idx3_v)
        copies = []
        for j in range(nchunk):
            for d in range(D_ART):
                copies.append(pltpu.async_copy(
                    emb_hbm.at[idx3_v.at[j, d]],
                    rows_v.at[d, pl.ds(j * 128, 128)], sem))
            idx = idx_v.at[j]
            lo = j * 128
            copies.append(pltpu.async_copy(
                lg_hbm.at[idx], g_v.at[pl.ds(lo, 128)], sem))
            copies.append(pltpu.async_copy(
                ls_hbm.at[idx], s_v.at[pl.ds(lo, 128)], sem))
            copies.append(pltpu.async_copy(
                lgr_hbm.at[idx], gr_v.at[pl.ds(lo, 128)], sem))
            copies.append(pltpu.async_copy(
                lc_hbm.at[idx], c_v.at[pl.ds(lo, 128)], sem))
        for cp in copies:
            cp.wait()
        pltpu.sync_copy(rows_v, emb_out.at[:, pl.ds(base, bpw)])
        pltpu.sync_copy(g_v, g_out.at[pl.ds(base, bpw)])
        pltpu.sync_copy(s_v, s_out.at[pl.ds(base, bpw)])
        pltpu.sync_copy(gr_v, gr_out.at[pl.ds(base, bpw)])
        pltpu.sync_copy(c_v, c_out.at[pl.ds(base, bpw)])

    emb_flat, nj = _tc_flatten(emb_table)
    d = jnp.arange(D_ART, dtype=jnp.int32)
    c_d = (d // 8) * (8 * nj * FLAT_L) + (d % 8) * FLAT_L
    t_a = (article_id // FLAT_L) * (8 * FLAT_L) + article_id % FLAT_L
    idx3 = (t_a.reshape(B // 128, 1, 128) + c_d.reshape(1, D_ART, 1))
    return k(article_id.reshape(B // 128, 128), idx3, emb_flat,
             lut_g, lut_s, lut_gr, lut_c)


def _prep_kernel(gt_ref, st_ref, grt_ref, ct_ref,
                 gam_ref, bet_ref, mu_ref, var_ref, w_ref,
                 w1_ref, p_ref, bias_ref):
    gam = gam_ref[:, :]
    a = gam * lax.rsqrt(var_ref[:, :] + EPS)
    bcol = bet_ref[:, :] - mu_ref[:, :] * a
    w = w_ref[:, :]
    wp = w * a
    bias_ref[:, :] = jnp.sum(w * bcol, axis=0, keepdims=True)
    w1_ref[:, :] = wp[0:D_ART, :]
    o0 = D_ART
    pg = jnp.dot(gt_ref[:, :], wp[o0:o0 + D_GRP, :],
                 preferred_element_type=jnp.float32)
    o1 = o0 + D_GRP
    pgr = jnp.dot(grt_ref[:, :], wp[o1:o1 + D_GRA, :],
                  preferred_element_type=jnp.float32)
    o2 = o1 + D_GRA
    pc = jnp.dot(ct_ref[:, :], wp[o2:o2 + D_COL, :],
                 preferred_element_type=jnp.float32)
    o3 = o2 + D_COL
    ps = jnp.dot(st_ref[:, :], wp[o3:o3 + D_SEC, :],
                 preferred_element_type=jnp.float32)
    pad = jnp.zeros((P_PAD - P_ROWS, 128), jnp.float32)
    p_ref[:, :] = jnp.concatenate([pg, pgr, pc, ps, pad], axis=0)


def _main_kernel(emb_ref, g_ref, s_ref, gr_ref, c_ref,
                 w1_ref, p_ref, bias_ref, out_ref):
    bb = emb_ref.shape[1]
    gid = g_ref[0]
    sid = s_ref[0]
    grid = gr_ref[0]
    cid = c_ref[0]
    col = lax.broadcasted_iota(jnp.int32, (bb, P_PAD), 1)
    oh = ((col == gid).astype(jnp.float32)
          + (col == grid + OFF_GR).astype(jnp.float32)
          + (col == cid + OFF_C).astype(jnp.float32)
          + (col == sid + OFF_S).astype(jnp.float32))
    acc = lax.dot_general(emb_ref[:, :], w1_ref[:, :],
                          dimension_numbers=(((0,), (0,)), ((), ())),
                          preferred_element_type=jnp.float32)
    acc = acc + jnp.dot(oh, p_ref[:, :], preferred_element_type=jnp.float32)
    out_ref[:, :] = acc + bias_ref[:, :]


def kernel(article_id, emb_table, group_table, section_table, graphical_table,
           colour_table, lut_group, lut_section, lut_graphical, lut_colour,
           gamma, beta, moving_mean, moving_var, W):
    embt, gid, sid, grid, cid = _sc_gather(
        article_id, emb_table,
        lut_group, lut_section, lut_graphical, lut_colour)

    cdim = W.shape[0]
    w1, p, bias = pl.pallas_call(
        _prep_kernel,
        out_shape=[
            jax.ShapeDtypeStruct((D_ART, 128), jnp.float32),
            jax.ShapeDtypeStruct((P_PAD, 128), jnp.float32),
            jax.ShapeDtypeStruct((1, 128), jnp.float32),
        ],
    )(group_table, section_table, graphical_table, colour_table,
      gamma.reshape(cdim, 1), beta.reshape(cdim, 1),
      moving_mean.reshape(cdim, 1), moving_var.reshape(cdim, 1), W)

    bb = 512
    nblk = B // bb
    ids_spec = pl.BlockSpec((1, bb, 1), lambda i: (i, 0, 0))
    full = lambda shape: pl.BlockSpec(shape, lambda i: (0,) * len(shape))
    out = pl.pallas_call(
        _main_kernel,
        grid=(nblk,),
        in_specs=[
            pl.BlockSpec((D_ART, bb), lambda i: (0, i)),
            ids_spec, ids_spec, ids_spec, ids_spec,
            full((D_ART, 128)), full((P_PAD, 128)), full((1, 128)),
        ],
        out_specs=pl.BlockSpec((bb, 128), lambda i: (i, 0)),
        out_shape=jax.ShapeDtypeStruct((B, 128), jnp.float32),
    )(embt,
      gid.reshape(nblk, bb, 1), sid.reshape(nblk, bb, 1),
      grid.reshape(nblk, bb, 1), cid.reshape(nblk, bb, 1),
      w1, p, bias)
    return out


# speedup vs baseline: 13.1962x; 1.0128x over previous
import functools

import jax
import jax.numpy as jnp
from jax import lax
from jax.experimental import pallas as pl
from jax.experimental.pallas import tpu as pltpu
from jax.experimental.pallas import tpu_sc as plsc

B = 16384
D_ART = 32
EPS = 1e-3
N_G, N_GR, N_C, N_S = 21, 31, 21, 57
OFF_GR = N_G
OFF_C = N_G + N_GR
OFF_S = N_G + N_GR + N_C
P_ROWS = N_G + N_GR + N_C + N_S
P_PAD = 136
D_GRP, D_GRA, D_COL, D_SEC = 10, 15, 10, 14


FLAT_L = 262144


def _flatten_kernel(in_ref, out_ref):
    blk = in_ref[...]
    out_ref[...] = jnp.reshape(blk, (blk.shape[0] * blk.shape[1],))


def _tc_flatten(emb_table):
    v, d_art = emb_table.shape
    nj = -(-v // FLAT_L)
    flat = pl.pallas_call(
        _flatten_kernel,
        grid=(d_art // 8, nj),
        in_specs=[pl.BlockSpec((8, FLAT_L), lambda g, j: (g, j))],
        out_specs=pl.BlockSpec((8 * FLAT_L,), lambda g, j: (g * nj + j,)),
        out_shape=jax.ShapeDtypeStruct((d_art * nj * FLAT_L,), jnp.float32),
    )(emb_table.T)
    return flat, nj


def _sc_gather(article_id, emb_table, lut_g, lut_s, lut_gr, lut_c):
    info = plsc.get_sparse_core_info()
    nc, ns = info.num_cores, info.num_subcores
    nw = nc * ns
    bpw = B // nw
    nchunk = bpw // 128

    mesh = plsc.VectorSubcoreMesh(core_axis_name="c", subcore_axis_name="s",
                                  num_cores=nc)

    @functools.partial(
        pl.kernel,
        mesh=mesh,
        out_type=[
            jax.ShapeDtypeStruct((D_ART, B), jnp.float32),
            jax.ShapeDtypeStruct((B,), jnp.int32),
            jax.ShapeDtypeStruct((B,), jnp.int32),
            jax.ShapeDtypeStruct((B,), jnp.int32),
            jax.ShapeDtypeStruct((B,), jnp.int32),
        ],
        scratch_types=[
            pltpu.VMEM((nchunk, 128), jnp.int32),
            pltpu.VMEM((nchunk, D_ART, 128), jnp.int32),
            pltpu.VMEM((D_ART, bpw), jnp.float32),
            pltpu.VMEM((bpw,), jnp.int32),
            pltpu.VMEM((bpw,), jnp.int32),
            pltpu.VMEM((bpw,), jnp.int32),
            pltpu.VMEM((bpw,), jnp.int32),
            pltpu.SemaphoreType.DMA,
        ],
        compiler_params=pltpu.CompilerParams(use_tc_tiling_on_sc=False),
    )
    def k(aid_hbm, idx3_hbm, emb_hbm, lg_hbm, ls_hbm, lgr_hbm, lc_hbm,
          emb_out, g_out, s_out, gr_out, c_out,
          idx_v, idx3_v, rows_v, g_v, s_v, gr_v, c_v, sem):
        wid = lax.axis_index("s") * nc + lax.axis_index("c")
        base = wid * bpw
        pltpu.sync_copy(aid_hbm.at[pl.ds(wid * nchunk, nchunk), :], idx_v)
        pltpu.sync_copy(idx3_hbm.at[pl.ds(wid * nchunk, nchunk), :, :], idx3_v)
        copies = []
        for j in range(nchunk):
            for d in range(D_ART):
                copies.append(pltpu.async_copy(
                    emb_hbm.at[idx3_v.at[j, d]],
                    rows_v.at[d, pl.ds(j * 128, 128)], sem))
            idx = idx_v.at[j]
            lo = j * 128
            copies.append(pltpu.async_copy(
                lg_hbm.at[idx], g_v.at[pl.ds(lo, 128)], sem))
            copies.append(pltpu.async_copy(
                ls_hbm.at[idx], s_v.at[pl.ds(lo, 128)], sem))
            copies.append(pltpu.async_copy(
                lgr_hbm.at[idx], gr_v.at[pl.ds(lo, 128)], sem))
            copies.append(pltpu.async_copy(
                lc_hbm.at[idx], c_v.at[pl.ds(lo, 128)], sem))
        for cp in copies:
            cp.wait()
        pltpu.sync_copy(rows_v, emb_out.at[:, pl.ds(base, bpw)])
        pltpu.sync_copy(g_v, g_out.at[pl.ds(base, bpw)])
        pltpu.sync_copy(s_v, s_out.at[pl.ds(base, bpw)])
        pltpu.sync_copy(gr_v, gr_out.at[pl.ds(base, bpw)])
        pltpu.sync_copy(c_v, c_out.at[pl.ds(base, bpw)])

    emb_flat, nj = _tc_flatten(emb_table)
    d = jnp.arange(D_ART, dtype=jnp.int32)
    c_d = (d // 8) * (8 * nj * FLAT_L) + (d % 8) * FLAT_L
    t_a = (article_id // FLAT_L) * (8 * FLAT_L) + article_id % FLAT_L
    idx3 = (t_a.reshape(B // 128, 1, 128) + c_d.reshape(1, D_ART, 1))
    return k(article_id.reshape(B // 128, 128), idx3, emb_flat,
             lut_g, lut_s, lut_gr, lut_c)


def _prep_kernel(gt_ref, st_ref, grt_ref, ct_ref,
                 gam_ref, bet_ref, mu_ref, var_ref, w_ref,
                 w1_ref, p_ref, bias_ref):
    gam = gam_ref[:, :]
    a = gam * lax.rsqrt(var_ref[:, :] + EPS)
    bcol = bet_ref[:, :] - mu_ref[:, :] * a
    w = w_ref[:, :]
    wp = w * a
    bias_ref[:, :] = jnp.sum(w * bcol, axis=0, keepdims=True)
    w1_ref[:, :] = wp[0:D_ART, :]
    o0 = D_ART
    pg = jnp.dot(gt_ref[:, :], wp[o0:o0 + D_GRP, :],
                 preferred_element_type=jnp.float32)
    o1 = o0 + D_GRP
    pgr = jnp.dot(grt_ref[:, :], wp[o1:o1 + D_GRA, :],
                  preferred_element_type=jnp.float32)
    o2 = o1 + D_GRA
    pc = jnp.dot(ct_ref[:, :], wp[o2:o2 + D_COL, :],
                 preferred_element_type=jnp.float32)
    o3 = o2 + D_COL
    ps = jnp.dot(st_ref[:, :], wp[o3:o3 + D_SEC, :],
                 preferred_element_type=jnp.float32)
    pad = jnp.zeros((P_PAD - P_ROWS, 128), jnp.float32)
    p_ref[:, :] = jnp.concatenate([pg, pgr, pc, ps, pad], axis=0)


def _main_kernel(emb_ref, g_ref, s_ref, gr_ref, c_ref,
                 w1_ref, p_ref, bias_ref, out_ref):
    bb = emb_ref.shape[1]
    gid = g_ref[0]
    sid = s_ref[0]
    grid = gr_ref[0]
    cid = c_ref[0]
    col = lax.broadcasted_iota(jnp.int32, (bb, P_PAD), 1)
    oh = ((col == gid).astype(jnp.float32)
          + (col == grid + OFF_GR).astype(jnp.float32)
          + (col == cid + OFF_C).astype(jnp.float32)
          + (col == sid + OFF_S).astype(jnp.float32))
    acc = lax.dot_general(emb_ref[:, :], w1_ref[:, :],
                          dimension_numbers=(((0,), (0,)), ((), ())),
                          preferred_element_type=jnp.float32)
    acc = acc + jnp.dot(oh, p_ref[:, :], preferred_element_type=jnp.float32)
    out_ref[:, :] = acc + bias_ref[:, :]


def kernel(article_id, emb_table, group_table, section_table, graphical_table,
           colour_table, lut_group, lut_section, lut_graphical, lut_colour,
           gamma, beta, moving_mean, moving_var, W):
    embt, gid, sid, grid, cid = _sc_gather(
        article_id, emb_table,
        lut_group, lut_section, lut_graphical, lut_colour)

    cdim = W.shape[0]
    w1, p, bias = pl.pallas_call(
        _prep_kernel,
        out_shape=[
            jax.ShapeDtypeStruct((D_ART, 128), jnp.float32),
            jax.ShapeDtypeStruct((P_PAD, 128), jnp.float32),
            jax.ShapeDtypeStruct((1, 128), jnp.float32),
        ],
    )(group_table, section_table, graphical_table, colour_table,
      gamma.reshape(cdim, 1), beta.reshape(cdim, 1),
      moving_mean.reshape(cdim, 1), moving_var.reshape(cdim, 1), W)

    bb = 512
    nblk = B // bb
    ids_spec = pl.BlockSpec((1, bb, 1), lambda i: (i, 0, 0))
    full = lambda shape: pl.BlockSpec(shape, lambda i: (0,) * len(shape))
    out = pl.pallas_call(
        _main_kernel,
        grid=(nblk,),
        in_specs=[
            pl.BlockSpec((D_ART, bb), lambda i: (0, i)),
            ids_spec, ids_spec, ids_spec, ids_spec,
            full((D_ART, 128)), full((P_PAD, 128)), full((1, 128)),
        ],
        out_specs=pl.BlockSpec((bb, 128), lambda i: (i, 0)),
        out_shape=jax.ShapeDtypeStruct((B, 128), jnp.float32),
    )(embt,
      gid.reshape(nblk, bb, 1), sid.reshape(nblk, bb, 1),
      grid.reshape(nblk, bb, 1), cid.reshape(nblk, bb, 1),
      w1, p, bias)
    return out
